# Initial kernel scaffold; baseline (speedup 1.0000x reference)
import jax
import jax.numpy as jnp
from jax.experimental import pallas as pl


def kernel(feats, pos, edge_index_g, edge_index_mp1, edge_index_mp2, W_adapt, b_adapt, W_T, W_A, ln_g, ln_b, W_mp1, W_mp2, Ws1, bs1, Ws2, Wc1, bc1, Wc2, bc2):
    raise NotImplementedError("write your pallas kernel here")


# speedup vs baseline: 1.1115x; 1.1115x over previous
import functools

import jax
import jax.numpy as jnp
from jax.experimental import pallas as pl

_N = 10000
_D = 128
_TAU = 0.8
_LAM = 0.5
_ALPHA = 1.0
_BETA = 1.0


def _pos_tile_kernel(n, bj, pos_ref, p1i_ref, p2i_ref, p1j_ref, p2j_ref,
                     deg_ref, r_ref, a_ref, c_ref, b_ref):
    j = pl.program_id(1)
    col0 = j * bj
    col_ids = col0 + jax.lax.broadcasted_iota(jnp.int32, (1, bj), 1)
    valid = col_ids < n

    pos_t = jnp.where(valid, pos_ref[...], 0.0)
    p1i = p1i_ref[...]
    p2i = p2i_ref[...]
    p1j = p1j_ref[...]
    p2j = p2j_ref[...]

    dn = (((1,), (1,)), ((), ()))
    sim_ij = jnp.exp(
        jax.lax.dot_general(p1i, p2j, dn, preferred_element_type=jnp.float32)
        / _TAU)
    sim_ti = jnp.exp(
        jax.lax.dot_general(p2i, p1j, dn, preferred_element_type=jnp.float32)
        / _TAU)
    sim_ij = jnp.where(valid, sim_ij, 0.0)
    sim_ti = jnp.where(valid, sim_ti, 0.0)

    deg_c = jnp.sum(pos_t, axis=1, keepdims=True)
    r_c = jnp.sum(sim_ij, axis=1, keepdims=True)
    a_c = jnp.sum(sim_ij * pos_t, axis=1, keepdims=True)
    c_c = jnp.sum(sim_ti, axis=1, keepdims=True)
    b_c = jnp.sum(sim_ti * pos_t, axis=1, keepdims=True)

    @pl.when(j == 0)
    def _():
        deg_ref[...] = jnp.zeros_like(deg_ref)
        r_ref[...] = jnp.zeros_like(r_ref)
        a_ref[...] = jnp.zeros_like(a_ref)
        c_ref[...] = jnp.zeros_like(c_ref)
        b_ref[...] = jnp.zeros_like(b_ref)

    deg_ref[...] += deg_c
    r_ref[...] += r_c
    a_ref[...] += a_c
    c_ref[...] += c_c
    b_ref[...] += b_c


def _contrast_sums(pos, p1n, p2n, n, bi, bj):
    grid = (pl.cdiv(n, bi), pl.cdiv(n, bj))
    out_shape = [jax.ShapeDtypeStruct((n, 1), jnp.float32)] * 5
    kern = functools.partial(_pos_tile_kernel, n, bj)
    return pl.pallas_call(
        kern,
        grid=grid,
        in_specs=[
            pl.BlockSpec((bi, bj), lambda i, j: (i, j)),
            pl.BlockSpec((bi, _D), lambda i, j: (i, 0)),
            pl.BlockSpec((bi, _D), lambda i, j: (i, 0)),
            pl.BlockSpec((bj, _D), lambda i, j: (j, 0)),
            pl.BlockSpec((bj, _D), lambda i, j: (j, 0)),
        ],
        out_specs=[pl.BlockSpec((bi, 1), lambda i, j: (i, 0))] * 5,
        out_shape=out_shape,
    )(pos, p1n, p2n, p1n, p2n)


def _layernorm(x, g, b):
    mu = x.mean(-1, keepdims=True)
    var = ((x - mu) ** 2).mean(-1, keepdims=True)
    return (x - mu) / jnp.sqrt(var + 1e-5) * g + b


def _sce_loss(x, y, alpha=3):
    xn = x / (jnp.linalg.norm(x, axis=-1, keepdims=True) + 1e-8)
    yn = y / (jnp.linalg.norm(y, axis=-1, keepdims=True) + 1e-8)
    return jnp.mean((1.0 - (xn * yn).sum(-1)) ** alpha)


def _gcn(h, ei, W):
    src, dst = ei[0], ei[1]
    deg = jax.ops.segment_sum(jnp.ones((ei.shape[1],), dtype=h.dtype), dst,
                              num_segments=_N)
    deg = jnp.maximum(deg, 1.0)
    norm = 1.0 / jnp.sqrt(deg[src] * deg[dst])
    m = h[src] * norm[:, None]
    agg = jax.ops.segment_sum(m, dst, num_segments=_N)
    return jax.nn.elu(agg @ W)


def kernel(feats, pos, edge_index_g, edge_index_mp1, edge_index_mp2,
           W_adapt, b_adapt, W_T, W_A, ln_g, ln_b, W_mp1, W_mp2,
           Ws1, bs1, Ws2, Wc1, bc1, Wc2, bc2):
    h = jax.nn.gelu(feats @ W_adapt + b_adapt)

    src, dst = edge_index_g[0], edge_index_g[1]
    t_h = h @ W_T
    x_e = t_h[src] * h[dst]
    a_e = jnp.exp(x_e @ W_A)[:, 0]
    att = jax.ops.segment_sum(a_e, dst, num_segments=_N)
    a_norm = a_e / (att[dst] + 1e-8)
    m = h[src] * a_norm[:, None]
    h1 = jax.ops.segment_sum(m, dst, num_segments=_N)
    h1 = _layernorm(jax.nn.elu(h1), ln_g, ln_b)

    z1 = _gcn(h, edge_index_mp1, W_mp1)
    z2 = _gcn(h, edge_index_mp2, W_mp2)

    zstack = jnp.stack([z1, z2], axis=1)
    w = (jnp.tanh(zstack @ Ws1 + bs1) @ Ws2).mean(0)
    beta_w = jax.nn.softmax(w, axis=0)
    h2 = (beta_w[None, :, :] * zstack).sum(1)

    def proj(x):
        return jax.nn.elu(x @ Wc1 + bc1) @ Wc2 + bc2

    p1 = proj(h1)
    p2 = proj(h2)
    p1n = p1 / (jnp.linalg.norm(p1, axis=1, keepdims=True) + 1e-8)
    p2n = p2 / (jnp.linalg.norm(p2, axis=1, keepdims=True) + 1e-8)

    deg, r, a, c, b = _contrast_sums(pos, p1n, p2n, _N, 1024, 1024)
    degree = deg[:, 0]

    k = _N // 2
    _, idx = jax.lax.top_k(degree, k)
    intra_loss = _sce_loss(z1[idx], z2[idx])

    l1 = -jnp.log(a[:, 0] / (r[:, 0] + 1e-8) + 1e-8).mean()
    l2 = -jnp.log(b[:, 0] / (c[:, 0] + 1e-8) + 1e-8).mean()
    inter_loss = _LAM * l1 + (1.0 - _LAM) * l2
    return _ALPHA * inter_loss + _BETA * intra_loss


# speedup vs baseline: 2.9808x; 2.6819x over previous
import functools

import jax
import jax.numpy as jnp
from jax import lax
from jax.experimental import pallas as pl
from jax.experimental.pallas import tpu as pltpu
from jax.experimental.pallas import tpu_sc as plsc

_N = 10000
_D = 128
_TAU = 0.8
_LAM = 0.5
_ALPHA = 1.0
_BETA = 1.0


def _pos_tile_kernel(n, bj, pos_ref, p1i_ref, p2i_ref, p1j_ref, p2j_ref,
                     deg_ref, r_ref, a_ref, c_ref, b_ref):
    j = pl.program_id(1)
    col0 = j * bj
    col_ids = col0 + jax.lax.broadcasted_iota(jnp.int32, (1, bj), 1)
    valid = col_ids < n

    pos_t = jnp.where(valid, pos_ref[...], 0.0)
    p1i = p1i_ref[...]
    p2i = p2i_ref[...]
    p1j = p1j_ref[...]
    p2j = p2j_ref[...]

    dn = (((1,), (1,)), ((), ()))
    sim_ij = jnp.exp(
        jax.lax.dot_general(p1i, p2j, dn, preferred_element_type=jnp.float32)
        / _TAU)
    sim_ti = jnp.exp(
        jax.lax.dot_general(p2i, p1j, dn, preferred_element_type=jnp.float32)
        / _TAU)
    sim_ij = jnp.where(valid, sim_ij, 0.0)
    sim_ti = jnp.where(valid, sim_ti, 0.0)

    deg_c = jnp.sum(pos_t, axis=1, keepdims=True)
    r_c = jnp.sum(sim_ij, axis=1, keepdims=True)
    a_c = jnp.sum(sim_ij * pos_t, axis=1, keepdims=True)
    c_c = jnp.sum(sim_ti, axis=1, keepdims=True)
    b_c = jnp.sum(sim_ti * pos_t, axis=1, keepdims=True)

    @pl.when(j == 0)
    def _():
        deg_ref[...] = jnp.zeros_like(deg_ref)
        r_ref[...] = jnp.zeros_like(r_ref)
        a_ref[...] = jnp.zeros_like(a_ref)
        c_ref[...] = jnp.zeros_like(c_ref)
        b_ref[...] = jnp.zeros_like(b_ref)

    deg_ref[...] += deg_c
    r_ref[...] += r_c
    a_ref[...] += a_c
    c_ref[...] += c_c
    b_ref[...] += b_c


def _contrast_sums(pos, p1n, p2n, n, bi, bj):
    grid = (pl.cdiv(n, bi), pl.cdiv(n, bj))
    out_shape = [jax.ShapeDtypeStruct((n, 1), jnp.float32)] * 5
    kern = functools.partial(_pos_tile_kernel, n, bj)
    return pl.pallas_call(
        kern,
        grid=grid,
        in_specs=[
            pl.BlockSpec((bi, bj), lambda i, j: (i, j)),
            pl.BlockSpec((bi, _D), lambda i, j: (i, 0)),
            pl.BlockSpec((bi, _D), lambda i, j: (i, 0)),
            pl.BlockSpec((bj, _D), lambda i, j: (j, 0)),
            pl.BlockSpec((bj, _D), lambda i, j: (j, 0)),
        ],
        out_specs=[pl.BlockSpec((bi, 1), lambda i, j: (i, 0))] * 5,
        out_shape=out_shape,
    )(pos, p1n, p2n, p1n, p2n)


_E = 320000
_SC_CORES = 2
_SC_SUBCORES = 16
_CH = 128


def _gcn_pair_sc_body(hs1_ref, hs2_ref, s1_ref, d1_ref, s2_ref, d2_ref,
                      zrow_ref, out_ref, src_v, dst_v, rows_v, acc_sh, sem):
    c = lax.axis_index("c")
    s = lax.axis_index("s")

    n_rows = 1000
    row0 = s * n_rows

    @pl.when(s < 10)
    def _():
        pltpu.sync_copy(zrow_ref.at[pl.ds(0, n_rows)],
                        acc_sh.at[pl.ds(row0, n_rows)])

    plsc.subcore_barrier()

    e_per_sub = _E // _SC_SUBCORES
    base = s * e_per_sub
    n_chunks = e_per_sub // _CH
    rem = e_per_sub - n_chunks * _CH

    def run(src_hbm, dst_hbm, tab_hbm):
        def chunk(start, sz):
            pltpu.sync_copy(src_hbm.at[pl.ds(start, sz)],
                            src_v.at[pl.ds(0, sz)])
            pltpu.sync_copy(dst_hbm.at[pl.ds(start, sz)],
                            dst_v.at[pl.ds(0, sz)])
            pltpu.async_copy(tab_hbm.at[src_v.at[pl.ds(0, sz)]],
                             rows_v.at[pl.ds(0, sz)], sem).wait()
            pltpu.sync_copy(rows_v.at[pl.ds(0, sz)],
                            acc_sh.at[dst_v.at[pl.ds(0, sz)]], add=True)

        def body(i, carry):
            chunk(base + i * _CH, _CH)
            return carry

        lax.fori_loop(0, n_chunks, body, 0)
        if rem:
            chunk(base + n_chunks * _CH, rem)

    @pl.when(c == 0)
    def _():
        run(s1_ref, d1_ref, hs1_ref)

    @pl.when(c == 1)
    def _():
        run(s2_ref, d2_ref, hs2_ref)

    plsc.subcore_barrier()

    @pl.when(s < 10)
    def _():
        pltpu.sync_copy(acc_sh.at[pl.ds(row0, n_rows)],
                        out_ref.at[c, pl.ds(row0, n_rows)])


def _gcn_pair_sc(hs1, hs2, src1, dst1, src2, dst2):
    zrow = jnp.zeros((1000, _D), jnp.float32)
    mesh = plsc.VectorSubcoreMesh(core_axis_name="c", subcore_axis_name="s")
    f = pl.kernel(
        _gcn_pair_sc_body,
        out_type=jax.ShapeDtypeStruct((_SC_CORES, _N, _D), jnp.float32),
        mesh=mesh,
        scratch_types=[
            pltpu.VMEM((_CH,), jnp.int32),
            pltpu.VMEM((_CH,), jnp.int32),
            pltpu.VMEM((_CH, _D), jnp.float32),
            pltpu.VMEM_SHARED((_N, _D), jnp.float32),
            pltpu.SemaphoreType.DMA,
        ],
    )
    out = f(hs1, hs2, src1, dst1, src2, dst2, zrow)
    return out[0], out[1]


def _layernorm(x, g, b):
    mu = x.mean(-1, keepdims=True)
    var = ((x - mu) ** 2).mean(-1, keepdims=True)
    return (x - mu) / jnp.sqrt(var + 1e-5) * g + b


def _sce_loss(x, y, alpha=3):
    xn = x / (jnp.linalg.norm(x, axis=-1, keepdims=True) + 1e-8)
    yn = y / (jnp.linalg.norm(y, axis=-1, keepdims=True) + 1e-8)
    return jnp.mean((1.0 - (xn * yn).sum(-1)) ** alpha)


def _gcn(h, ei, W):
    src, dst = ei[0], ei[1]
    deg = jax.ops.segment_sum(jnp.ones((ei.shape[1],), dtype=h.dtype), dst,
                              num_segments=_N)
    deg = jnp.maximum(deg, 1.0)
    norm = 1.0 / jnp.sqrt(deg[src] * deg[dst])
    m = h[src] * norm[:, None]
    agg = jax.ops.segment_sum(m, dst, num_segments=_N)
    return jax.nn.elu(agg @ W)


def kernel(feats, pos, edge_index_g, edge_index_mp1, edge_index_mp2,
           W_adapt, b_adapt, W_T, W_A, ln_g, ln_b, W_mp1, W_mp2,
           Ws1, bs1, Ws2, Wc1, bc1, Wc2, bc2):
    h = jax.nn.gelu(feats @ W_adapt + b_adapt)

    src, dst = edge_index_g[0], edge_index_g[1]
    t_h = h @ W_T
    x_e = t_h[src] * h[dst]
    a_e = jnp.exp(x_e @ W_A)[:, 0]
    att = jax.ops.segment_sum(a_e, dst, num_segments=_N)
    a_norm = a_e / (att[dst] + 1e-8)
    m = h[src] * a_norm[:, None]
    h1 = jax.ops.segment_sum(m, dst, num_segments=_N)
    h1 = _layernorm(jax.nn.elu(h1), ln_g, ln_b)

    s1, d1 = edge_index_mp1[0], edge_index_mp1[1]
    s2, d2 = edge_index_mp2[0], edge_index_mp2[1]
    ones_e = jnp.ones((_E,), jnp.float32)
    deg1 = jnp.maximum(jax.ops.segment_sum(ones_e, d1, num_segments=_N), 1.0)
    deg2 = jnp.maximum(jax.ops.segment_sum(ones_e, d2, num_segments=_N), 1.0)
    di1 = jax.lax.rsqrt(deg1)
    di2 = jax.lax.rsqrt(deg2)
    agg1, agg2 = _gcn_pair_sc(h * di1[:, None], h * di2[:, None],
                              s1, d1, s2, d2)
    z1 = jax.nn.elu((agg1 * di1[:, None]) @ W_mp1)
    z2 = jax.nn.elu((agg2 * di2[:, None]) @ W_mp2)

    zstack = jnp.stack([z1, z2], axis=1)
    w = (jnp.tanh(zstack @ Ws1 + bs1) @ Ws2).mean(0)
    beta_w = jax.nn.softmax(w, axis=0)
    h2 = (beta_w[None, :, :] * zstack).sum(1)

    def proj(x):
        return jax.nn.elu(x @ Wc1 + bc1) @ Wc2 + bc2

    p1 = proj(h1)
    p2 = proj(h2)
    p1n = p1 / (jnp.linalg.norm(p1, axis=1, keepdims=True) + 1e-8)
    p2n = p2 / (jnp.linalg.norm(p2, axis=1, keepdims=True) + 1e-8)

    deg, r, a, c, b = _contrast_sums(pos, p1n, p2n, _N, 1024, 1024)
    degree = deg[:, 0]

    k = _N // 2
    _, idx = jax.lax.top_k(degree, k)
    intra_loss = _sce_loss(z1[idx], z2[idx])

    l1 = -jnp.log(a[:, 0] / (r[:, 0] + 1e-8) + 1e-8).mean()
    l2 = -jnp.log(b[:, 0] / (c[:, 0] + 1e-8) + 1e-8).mean()
    inter_loss = _LAM * l1 + (1.0 - _LAM) * l2
    return _ALPHA * inter_loss + _BETA * intra_loss
